# Initial kernel scaffold; baseline (speedup 1.0000x reference)
import jax
import jax.numpy as jnp
from jax.experimental import pallas as pl


def kernel(input_ids, embedding_table):
    raise NotImplementedError("write your pallas kernel here")


# speedup vs baseline: 6.6232x; 6.6232x over previous
import functools

import jax
import jax.numpy as jnp
from jax import lax
from jax.experimental import pallas as pl
from jax.experimental.pallas import tpu as pltpu
from jax.experimental.pallas import tpu_sc as plsc

_VOCAB = 100000
_D = 128
_MAX_LEN = 4096
_B = 1024
_L = 200

_NC, _NS = 2, 16
_NW = _NC * _NS
_N = _B * _L
_PER_W = _N // _NW
_CHUNK = _L
_NCHUNK = _PER_W // _CHUNK


def _pe_table():
    position = jnp.arange(_L)[:, None]
    i = jnp.arange(_D)[None, :]
    angles = position * (1.0 / jnp.power(10000.0, 2 * (i // 2) / _D))
    pe = jnp.zeros((_L, _D), dtype=jnp.float32)
    pe = pe.at[:, 0::2].set(jnp.sin(angles[:, 0::2]).astype(jnp.float32))
    pe = pe.at[:, 1::2].set(jnp.cos(angles[:, 1::2]).astype(jnp.float32))
    return pe


@functools.cache
def _build_sc_embed():
    @functools.partial(
        pl.kernel,
        out_type=jax.ShapeDtypeStruct((_N, _D), jnp.float32),
        mesh=plsc.VectorSubcoreMesh(
            core_axis_name="c", subcore_axis_name="s", num_cores=_NC, num_subcores=_NS
        ),
        scratch_types=[
            pltpu.VMEM((_PER_W,), jnp.int32),
            pltpu.VMEM((_CHUNK, _D), jnp.float32),
            pltpu.VMEM((_CHUNK, _D), jnp.float32),
            pltpu.VMEM((_CHUNK, _D), jnp.float32),
            pltpu.SemaphoreType.DMA,
            pltpu.SemaphoreType.DMA,
        ],
    )
    def _sc_embed(
        table_hbm, ids_hbm, pe_hbm, out_hbm, idx_v, rows0, rows1, pe_v, sem0, sem1
    ):
        wid = lax.axis_index("s") * _NC + lax.axis_index("c")
        base = wid * _PER_W
        pltpu.sync_copy(ids_hbm.at[pl.ds(base, _PER_W)], idx_v)
        pltpu.sync_copy(pe_hbm, pe_v)
        bufs = ((rows0, sem0), (rows1, sem1))

        def gather_parts(k, buf, sem):
            p1 = pltpu.make_async_copy(
                table_hbm.at[idx_v.at[pl.ds(k * _CHUNK, 104)]],
                buf.at[pl.ds(0, 104)],
                sem,
            )
            p2 = pltpu.make_async_copy(
                table_hbm.at[idx_v.at[pl.ds(k * _CHUNK + 104, 96)]],
                buf.at[pl.ds(104, 96)],
                sem,
            )
            return p1, p2

        def start(k, buf, sem):
            p1, p2 = gather_parts(k, buf, sem)
            p1.start()
            p2.start()

        start(0, rows0, sem0)
        start(1, rows1, sem1)

        def pair_body(m, _):
            for b in range(2):
                k = 2 * m + b
                buf, sem = bufs[b]
                p1, p2 = gather_parts(k, buf, sem)
                p1.wait()
                p2.wait()

                def row_body(i, _):
                    for j in range(_D // 16):
                        plsc.addupdate(
                            buf.at[i, pl.ds(j * 16, 16)], pe_v[i, pl.ds(j * 16, 16)]
                        )
                    return 0

                lax.fori_loop(0, _CHUNK, row_body, 0)
                pltpu.sync_copy(buf, out_hbm.at[pl.ds(base + k * _CHUNK, _CHUNK)])

                @pl.when(k + 2 < _NCHUNK)
                def _():
                    start(k + 2, buf, sem)

            return 0

        lax.fori_loop(0, _NCHUNK // 2, pair_body, 0)

    return _sc_embed


def kernel(input_ids, embedding_table):
    flat_ids = input_ids.reshape(-1).astype(jnp.int32)
    pe = _pe_table()
    out = _build_sc_embed()(embedding_table, flat_ids, pe)
    return out.reshape(_B, _L, _D)


# speedup vs baseline: 7.6531x; 1.1555x over previous
import functools

import jax
import jax.numpy as jnp
from jax import lax
from jax.experimental import pallas as pl
from jax.experimental.pallas import tpu as pltpu
from jax.experimental.pallas import tpu_sc as plsc

_VOCAB = 100000
_D = 128
_MAX_LEN = 4096
_B = 1024
_L = 200

_NC, _NS = 2, 16
_NW = _NC * _NS
_N = _B * _L
_PER_W = _N // _NW
_CHUNK = _L
_NCHUNK = _PER_W // _CHUNK


def _pe_table():
    position = jnp.arange(_L)[:, None]
    i = jnp.arange(_D)[None, :]
    angles = position * (1.0 / jnp.power(10000.0, 2 * (i // 2) / _D))
    pe = jnp.zeros((_L, _D), dtype=jnp.float32)
    pe = pe.at[:, 0::2].set(jnp.sin(angles[:, 0::2]).astype(jnp.float32))
    pe = pe.at[:, 1::2].set(jnp.cos(angles[:, 1::2]).astype(jnp.float32))
    return pe


@functools.cache
def _build_sc_embed():
    @functools.partial(
        pl.kernel,
        out_type=jax.ShapeDtypeStruct((_N, _D), jnp.float32),
        mesh=plsc.VectorSubcoreMesh(
            core_axis_name="c", subcore_axis_name="s", num_cores=_NC, num_subcores=_NS
        ),
        scratch_types=[
            pltpu.VMEM((_PER_W,), jnp.int32),
            pltpu.VMEM((_CHUNK, _D), jnp.float32),
            pltpu.VMEM((_CHUNK, _D), jnp.float32),
            pltpu.VMEM((_CHUNK, _D), jnp.float32),
            pltpu.VMEM((_CHUNK, _D), jnp.float32),
            pltpu.SemaphoreType.DMA,
            pltpu.SemaphoreType.DMA,
            pltpu.SemaphoreType.DMA,
            pltpu.SemaphoreType.DMA,
            pltpu.SemaphoreType.DMA,
            pltpu.SemaphoreType.DMA,
        ],
    )
    def _sc_embed(
        table_hbm, ids_hbm, pe_hbm, out_hbm,
        idx_v, rows0, rows1, rows2, pe_v,
        g0, g1, g2, w0, w1, w2,
    ):
        wid = lax.axis_index("s") * _NC + lax.axis_index("c")
        base = wid * _PER_W
        pltpu.sync_copy(ids_hbm.at[pl.ds(base, _PER_W)], idx_v)
        pltpu.sync_copy(pe_hbm, pe_v)
        bufs = (rows0, rows1, rows2)
        gsems = (g0, g1, g2)
        wsems = (w0, w1, w2)

        def gather_parts(k, buf, sem):
            p1 = pltpu.make_async_copy(
                table_hbm.at[idx_v.at[pl.ds(k * _CHUNK, 104)]],
                buf.at[pl.ds(0, 104)],
                sem,
            )
            p2 = pltpu.make_async_copy(
                table_hbm.at[idx_v.at[pl.ds(k * _CHUNK + 104, 96)]],
                buf.at[pl.ds(104, 96)],
                sem,
            )
            return p1, p2

        def start_gather(k, b):
            p1, p2 = gather_parts(k, bufs[b], gsems[b])
            p1.start()
            p2.start()

        def wait_gather(k, b):
            p1, p2 = gather_parts(k, bufs[b], gsems[b])
            p1.wait()
            p2.wait()

        def write_desc(k, b):
            return pltpu.make_async_copy(
                bufs[b], out_hbm.at[pl.ds(base + k * _CHUNK, _CHUNK)], wsems[b]
            )

        def add_pe(b):
            buf = bufs[b]

            def row_body(i, _):
                for r in range(2):
                    for c in range(_D // 16):
                        plsc.addupdate(
                            buf.at[2 * i + r, pl.ds(c * 16, 16)],
                            pe_v[2 * i + r, pl.ds(c * 16, 16)],
                        )
                return 0

            lax.fori_loop(0, _CHUNK // 2, row_body, 0)

        def step(j, b, drain_prev, next_gather):
            wait_gather(j, b)
            add_pe(b)
            write_desc(j, b).start()
            pb = (b + 2) % 3
            if drain_prev:
                write_desc(j - 1, pb).wait()
            if next_gather:
                start_gather(j + 2, pb)

        start_gather(0, 0)
        start_gather(1, 1)
        step(0, 0, False, True)
        step(1, 1, True, True)
        step(2, 2, True, True)

        def ring_body(m, _):
            for b in range(3):
                step(3 * m + b, b, True, True)
            return 0

        lax.fori_loop(1, _NCHUNK // 3, ring_body, 0)

        step(30, 0, True, False)
        step(31, 1, True, False)
        write_desc(31, 1).wait()

    return _sc_embed


def kernel(input_ids, embedding_table):
    flat_ids = input_ids.reshape(-1).astype(jnp.int32)
    pe = _pe_table()
    out = _build_sc_embed()(embedding_table, flat_ids, pe)
    return out.reshape(_B, _L, _D)
